# baseline (speedup 1.0000x reference)
import functools
import math

import jax
import jax.numpy as jnp
from jax import lax
from jax.experimental import pallas as pl
from jax.experimental.pallas import tpu as pltpu
from jax.experimental.pallas import tpu_sc as plsc

_NC = 2
_NS = 16
_NW = _NC * _NS
_L = 16

_INT_MAX = jnp.iinfo(jnp.int32).max


def _bisect(keys_ref, q, num_keys, steps):
    lo = jnp.zeros((_L,), jnp.int32)
    hi = jnp.full((_L,), num_keys, jnp.int32)
    for _ in range(steps):
        mid = lax.shift_right_arithmetic(lo + hi, 1)
        kv = plsc.load_gather(keys_ref, [mid])
        pred = kv < q
        lo = jnp.where(pred, mid + 1, lo)
        hi = jnp.where(pred, hi, mid)
    idxc = jnp.minimum(lo, num_keys - 1)
    kv = plsc.load_gather(keys_ref, [idxc])
    return idxc, kv == q


def _make_sc_lookup(n, seq_len, vocab, c1, c2):
    chunk = n // _NW
    steps1 = max(1, math.ceil(math.log2(c1 + 1)))
    steps2 = max(1, math.ceil(math.log2(c2 + 1)))
    pad1 = c1 + _L
    pad2 = c2 + _L
    mesh = plsc.VectorSubcoreMesh(
        core_axis_name="c", subcore_axis_name="s",
        num_cores=_NC, num_subcores=_NS)
    out_sds = jax.ShapeDtypeStruct((n,), jnp.int32)

    @functools.partial(
        pl.kernel,
        out_type=(out_sds, out_sds, out_sds, out_sds, out_sds),
        mesh=mesh,
        compiler_params=pltpu.CompilerParams(needs_layout_passes=False),
        scratch_types=[
            pltpu.VMEM((chunk + 8,), jnp.int32),
            pltpu.VMEM((pad1,), jnp.int32),
            pltpu.VMEM((pad2,), jnp.int32),
            pltpu.VMEM((chunk,), jnp.int32),
            pltpu.VMEM((chunk,), jnp.int32),
            pltpu.VMEM((chunk,), jnp.int32),
            pltpu.VMEM((chunk,), jnp.int32),
            pltpu.VMEM((chunk,), jnp.int32),
        ],
    )
    def sc_lookup(batch_hbm, k1_hbm, k2_hbm,
                  sel_hbm, i1_hbm, i2_hbm, f1_hbm, f2_hbm,
                  qbuf, k1v, k2v, o_sel, o_i1, o_i2, o_f1, o_f2):
        wid = lax.axis_index("s") * _NC + lax.axis_index("c")
        base = wid * chunk
        pltpu.sync_copy(k1_hbm, k1v.at[pl.ds(0, c1)])
        pltpu.sync_copy(k2_hbm, k2v.at[pl.ds(0, c2)])
        k1v[pl.ds(c1, _L)] = jnp.full((_L,), _INT_MAX, jnp.int32)
        k2v[pl.ds(c2, _L)] = jnp.full((_L,), _INT_MAX, jnp.int32)
        pltpu.sync_copy(batch_hbm.at[pl.ds(base, chunk)],
                        qbuf.at[pl.ds(8, chunk)])
        @pl.when(wid > 0)
        def _():
            pltpu.sync_copy(batch_hbm.at[pl.ds(base - 8, 8)],
                            qbuf.at[pl.ds(0, 8)])

        iota = lax.iota(jnp.int32, _L)
        zeros = jnp.zeros((_L,), jnp.int32)
        c1vec = jnp.full((_L,), c1, jnp.int32)

        def step(t, carry):
            off = t * _L
            j = lax.rem(base + off, seq_len) + iota
            idxs = off + 8 + iota
            q1 = plsc.load_gather(qbuf, [idxs - 1])
            t2 = plsc.load_gather(qbuf, [idxs - 2])
            i1, hit1 = _bisect(k1v, q1, c1, steps1)
            q2 = t2 * vocab + q1
            i2, hit2 = _bisect(k2v, q2, c2, steps2)
            f1 = hit1 & (j >= 1)
            f2 = hit2 & (j >= 2)
            o_sel[pl.ds(off, _L)] = jnp.where(f1 & ~f2, i1, c1vec)
            o_i1[pl.ds(off, _L)] = jnp.where(f1 & f2, i1, zeros)
            o_i2[pl.ds(off, _L)] = jnp.where(f2, i2, zeros)
            o_f1[pl.ds(off, _L)] = f1.astype(jnp.int32)
            o_f2[pl.ds(off, _L)] = f2.astype(jnp.int32)
            return carry

        lax.fori_loop(0, chunk // _L, step, 0)
        pltpu.sync_copy(o_sel, sel_hbm.at[pl.ds(base, chunk)])
        pltpu.sync_copy(o_i1, i1_hbm.at[pl.ds(base, chunk)])
        pltpu.sync_copy(o_i2, i2_hbm.at[pl.ds(base, chunk)])
        pltpu.sync_copy(o_f1, f1_hbm.at[pl.ds(base, chunk)])
        pltpu.sync_copy(o_f2, f2_hbm.at[pl.ds(base, chunk)])

    return sc_lookup


_TR = 8


def _table_body(p0_ref, al_ref, rows_ref, out_ref, *, nsteps):
    i = pl.program_id(0)
    a0 = al_ref[0]
    a1 = al_ref[1]
    a2 = al_ref[2]
    vocab = out_ref.shape[1]
    base = a0 * p0_ref[0, :] + a2 * (1.0 / vocab)

    @pl.when(i < nsteps - 1)
    def _():
        out_ref[...] = jnp.log(base[None, :] + a1 * rows_ref[...])

    @pl.when(i == nsteps - 1)
    def _():
        row = jnp.log(base + a1 * (1.0 / vocab))
        out_ref[...] = jnp.broadcast_to(row[None, :], out_ref.shape)


def _make_tc_table(vocab, c1):
    nblk = c1 // _TR
    nsteps = nblk + 1
    grid_spec = pl.GridSpec(
        grid=(nsteps,),
        in_specs=[
            pl.BlockSpec((1, vocab), lambda i: (0, 0)),
            pl.BlockSpec(memory_space=pltpu.SMEM),
            pl.BlockSpec((_TR, vocab), lambda i: (jnp.minimum(i, nblk - 1), 0)),
        ],
        out_specs=pl.BlockSpec((_TR, vocab), lambda i: (i, 0)),
    )
    return pl.pallas_call(
        functools.partial(_table_body, nsteps=nsteps),
        grid_spec=grid_spec,
        out_shape=jax.ShapeDtypeStruct((c1 + _TR, vocab), jnp.float32),
        compiler_params=pltpu.CompilerParams(
            dimension_semantics=("arbitrary",)),
    )


_P = 16


def _tc_body(sels, i1s, i2s, f1s, f2s, p0_ref, al_ref, *refs):
    trows = refs[:_P]
    rows1 = refs[_P:2 * _P]
    rows2 = refs[2 * _P:3 * _P]
    out_ref = refs[3 * _P]
    g = pl.program_id(0)
    a0 = al_ref[0]
    a1 = al_ref[1]
    a2 = al_ref[2]
    vocab = out_ref.shape[1]
    base = a0 * p0_ref[0, :]
    for k in range(_P):
        p = g * _P + k
        f2 = f2s[p]

        @pl.when(f2 == 0)
        def _(k=k):
            out_ref[k, :] = trows[k][0, 0, :]

        @pl.when(f2 != 0)
        def _(k=k):
            c1c = a1 * f1s[p].astype(jnp.float32)
            miss = (a1 - c1c) * (1.0 / vocab)
            out_ref[k, :] = jnp.log(
                (base + miss) + c1c * rows1[k][0, 0, :]
                + a2 * rows2[k][0, 0, :])


def _make_tc_combine(n, vocab, c1, c2):
    def imt(k, g, sel, i1, i2, f1, f2):
        return (sel[g * _P + k], 0, 0)

    def im1(k, g, sel, i1, i2, f1, f2):
        return (i1[g * _P + k], 0, 0)

    def im2(k, g, sel, i1, i2, f1, f2):
        return (i2[g * _P + k], 0, 0)

    grid_spec = pltpu.PrefetchScalarGridSpec(
        num_scalar_prefetch=5,
        grid=(n // _P,),
        in_specs=[
            pl.BlockSpec((1, vocab), lambda g, *s: (0, 0)),
            pl.BlockSpec(memory_space=pltpu.SMEM),
            *[pl.BlockSpec((1, 1, vocab), functools.partial(imt, k))
              for k in range(_P)],
            *[pl.BlockSpec((1, 1, vocab), functools.partial(im1, k))
              for k in range(_P)],
            *[pl.BlockSpec((1, 1, vocab), functools.partial(im2, k))
              for k in range(_P)],
        ],
        out_specs=pl.BlockSpec((_P, vocab), lambda g, *s: (g, 0)),
    )
    return pl.pallas_call(
        _tc_body,
        grid_spec=grid_spec,
        out_shape=jax.ShapeDtypeStruct((n, vocab), jnp.float32),
        compiler_params=pltpu.CompilerParams(
            dimension_semantics=("arbitrary",)),
    )


def kernel(batch, alphas, p0, ctx1_keys, ctx1_probs, ctx2_keys, ctx2_probs):
    b, s = batch.shape
    vocab = p0.shape[0]
    c1 = ctx1_keys.shape[0]
    c2 = ctx2_keys.shape[0]
    n = b * s
    flat = batch.reshape(n).astype(jnp.int32)
    p0_2d = p0.reshape(1, vocab)

    sc_lookup = _make_sc_lookup(n, s, vocab, c1, c2)
    sel, i1, i2, f1, f2 = sc_lookup(flat, ctx1_keys, ctx2_keys)

    table = _make_tc_table(vocab, c1)(p0_2d, alphas, ctx1_probs)

    tc_combine = _make_tc_combine(n, vocab, c1, c2)
    tr = table.reshape(-1, 1, vocab)
    r1 = ctx1_probs.reshape(c1, 1, vocab)
    r2 = ctx2_probs.reshape(c2, 1, vocab)
    out = tc_combine(sel, i1, i2, f1, f2, p0_2d, alphas,
                     *([tr] * _P), *([r1] * _P), *([r2] * _P))
    return out.reshape(b, s, vocab)


# speedup vs baseline: 1.7351x; 1.7351x over previous
import functools
import math

import jax
import jax.numpy as jnp
from jax import lax
from jax.experimental import pallas as pl
from jax.experimental.pallas import tpu as pltpu
from jax.experimental.pallas import tpu_sc as plsc

_NC = 2
_NS = 16
_NW = _NC * _NS
_L = 16

_INT_MAX = jnp.iinfo(jnp.int32).max
_LN2 = 0.6931471805599453


def _bisect(keys_ref, q, num_keys, steps):
    lo = jnp.zeros((_L,), jnp.int32)
    hi = jnp.full((_L,), num_keys, jnp.int32)
    for _ in range(steps):
        mid = lax.shift_right_arithmetic(lo + hi, 1)
        kv = plsc.load_gather(keys_ref, [mid])
        pred = kv < q
        lo = jnp.where(pred, mid + 1, lo)
        hi = jnp.where(pred, hi, mid)
    idxc = jnp.minimum(lo, num_keys - 1)
    kv = plsc.load_gather(keys_ref, [idxc])
    return idxc, kv == q


def _vlog(x):
    bits = plsc.bitcast(x, jnp.int32)
    e = (lax.shift_right_logical(bits, 23) & 0xFF) - 127
    m = plsc.bitcast((bits & 0x7FFFFF) | 0x3F800000, jnp.float32)
    big = m > (4.0 / 3.0)
    m = jnp.where(big, m * 0.5, m)
    e = jnp.where(big, e + 1, e)
    r = m - 1.0
    s = jnp.float32(-1.0 / 14.0)
    for kk in range(13, 0, -1):
        c = jnp.float32((1.0 if kk % 2 else -1.0) / kk)
        s = s * r + c
    s = s * r
    return e.astype(jnp.float32) * jnp.float32(_LN2) + s


_G = 32


def _make_sc_stage(n, seq_len, vocab, c1, c2):
    chunk = n // _NW
    steps1 = max(1, math.ceil(math.log2(c1 + 1)))
    steps2 = max(1, math.ceil(math.log2(c2 + 1)))
    vp128 = (vocab + 127) // 128 * 128
    vpad = vp128
    nvec = vpad // _L
    mesh = plsc.VectorSubcoreMesh(
        core_axis_name="c", subcore_axis_name="s",
        num_cores=_NC, num_subcores=_NS)

    @functools.partial(
        pl.kernel,
        out_type=jax.ShapeDtypeStruct((n, vp128), jnp.float32),
        mesh=mesh,
        compiler_params=pltpu.CompilerParams(
            needs_layout_passes=False, use_tc_tiling_on_sc=False),
        scratch_types=[
            pltpu.VMEM((chunk + 8,), jnp.int32),
            pltpu.VMEM((c1 + _L,), jnp.int32),
            pltpu.VMEM((c2 + _L,), jnp.int32),
            pltpu.VMEM((chunk,), jnp.int32),
            pltpu.VMEM((chunk,), jnp.int32),
            pltpu.VMEM((_G, vp128), jnp.float32),
            pltpu.VMEM((vpad,), jnp.float32),
            pltpu.VMEM((vpad,), jnp.float32),
            pltpu.VMEM((vpad,), jnp.float32),
            pltpu.VMEM((vpad,), jnp.float32),
            pltpu.VMEM((_L,), jnp.float32),
            pltpu.SemaphoreType.DMA,
        ],
    )
    def sc_stage(batch_hbm, k1_hbm, k2_hbm, t_hbm, p0_hbm, p1_hbm, p2_hbm,
                 al_hbm, out_hbm,
                 qbuf, k1v, k2v, selv, encv, gbuf, p0v, r1v, r2v, rowo,
                 alv, sem):
        wid = lax.axis_index("s") * _NC + lax.axis_index("c")
        base = wid * chunk
        pltpu.sync_copy(k1_hbm, k1v.at[pl.ds(0, c1)])
        pltpu.sync_copy(k2_hbm, k2v.at[pl.ds(0, c2)])
        k1v[pl.ds(c1, _L)] = jnp.full((_L,), _INT_MAX, jnp.int32)
        k2v[pl.ds(c2, _L)] = jnp.full((_L,), _INT_MAX, jnp.int32)
        pltpu.sync_copy(al_hbm, alv)
        pltpu.sync_copy(p0_hbm, p0v.at[pl.ds(0, vocab)])
        pltpu.sync_copy(batch_hbm.at[pl.ds(base, chunk)],
                        qbuf.at[pl.ds(8, chunk)])
        @pl.when(wid > 0)
        def _():
            pltpu.sync_copy(batch_hbm.at[pl.ds(base - 8, 8)],
                            qbuf.at[pl.ds(0, 8)])

        iota = lax.iota(jnp.int32, _L)
        c1vec = jnp.full((_L,), c1, jnp.int32)

        def lookup_step(t, carry):
            off = t * _L
            j = lax.rem(base + off, seq_len) + iota
            idxs = off + 8 + iota
            q1 = plsc.load_gather(qbuf, [idxs - 1])
            t2 = plsc.load_gather(qbuf, [idxs - 2])
            i1, hit1 = _bisect(k1v, q1, c1, steps1)
            q2 = t2 * vocab + q1
            i2, hit2 = _bisect(k2v, q2, c2, steps2)
            f1 = hit1 & (j >= 1)
            f2 = hit2 & (j >= 2)
            selv[pl.ds(off, _L)] = jnp.where(f1 & ~f2, i1, c1vec)
            i1p = jnp.where(f1, i1 + 1, 0)
            enc = (lax.shift_left(i2, 12) | lax.shift_left(i1p, 1)
                   | jnp.where(f2, 1, 0))
            encv[pl.ds(off, _L)] = jnp.where(f2, enc, 0)
            return carry

        lax.fori_loop(0, chunk // _L, lookup_step, 0)

        def gather_step(c, carry):
            off = c * _G
            pltpu.async_copy(t_hbm.at[selv.at[pl.ds(off, _G)]],
                             gbuf, sem).wait()
            pltpu.sync_copy(gbuf, out_hbm.at[pl.ds(base + off, _G)])
            return carry

        lax.fori_loop(0, chunk // _G, gather_step, 0)

        av = alv[...]
        fzero = jnp.float32(0.0)
        a0 = jnp.max(jnp.where(iota == 0, av, fzero))
        a1 = jnp.max(jnp.where(iota == 1, av, fzero))
        a2 = jnp.max(jnp.where(iota == 2, av, fzero))
        uni = jnp.float32(1.0 / vocab)
        izero = jnp.zeros((_L,), jnp.int32)

        def fix_group(g, carry):
            encg = encv[pl.ds(g * _L, _L)]

            @pl.when(jnp.max(encg) != 0)
            def _():
                def fix_lane(lane, carry2):
                    enc = jnp.max(jnp.where(iota == lane, encg, izero))

                    @pl.when(enc != 0)
                    def _():
                        i1p = lax.shift_right_logical(enc, 1) & 0x7FF
                        i2 = lax.shift_right_logical(enc, 12)
                        i1idx = jnp.maximum(i1p - 1, 0)
                        pltpu.sync_copy(p1_hbm.at[i1idx],
                                        r1v.at[pl.ds(0, vocab)])
                        pltpu.sync_copy(p2_hbm.at[i2],
                                        r2v.at[pl.ds(0, vocab)])
                        c1c = jnp.where(i1p > 0, a1, fzero)
                        missc = (a1 - c1c) * uni

                        def vec_step(v, carry3):
                            ds = pl.ds(v * _L, _L)
                            x = (a0 * p0v[ds] + missc + c1c * r1v[ds]
                                 + a2 * r2v[ds])
                            rowo[ds] = _vlog(x)
                            return carry3

                        lax.fori_loop(0, nvec, vec_step, 0)
                        pltpu.sync_copy(rowo,
                                        out_hbm.at[base + g * _L + lane])

                    return carry2

                lax.fori_loop(0, _L, fix_lane, 0)

            return carry

        lax.fori_loop(0, chunk // _L, fix_group, 0)

    return sc_stage


_TR = 8


def _table_body(p0_ref, al_ref, rows_ref, out_ref, *, nsteps, uni):
    i = pl.program_id(0)
    a0 = al_ref[0]
    a1 = al_ref[1]
    a2 = al_ref[2]
    base = a0 * p0_ref[0, :] + a2 * uni

    @pl.when(i < nsteps - 1)
    def _():
        out_ref[...] = jnp.log(base[None, :] + a1 * rows_ref[...])

    @pl.when(i == nsteps - 1)
    def _():
        row = jnp.log(base + a1 * uni)
        out_ref[...] = jnp.broadcast_to(row[None, :], out_ref.shape)


def _make_tc_table(vocab, c1, true_vocab):
    nblk = c1 // _TR
    nsteps = nblk + 1
    grid_spec = pl.GridSpec(
        grid=(nsteps,),
        in_specs=[
            pl.BlockSpec((1, vocab), lambda i: (0, 0)),
            pl.BlockSpec(memory_space=pltpu.SMEM),
            pl.BlockSpec((_TR, vocab), lambda i: (jnp.minimum(i, nblk - 1), 0)),
        ],
        out_specs=pl.BlockSpec((_TR, vocab), lambda i: (i, 0)),
    )
    return pl.pallas_call(
        functools.partial(_table_body, nsteps=nsteps, uni=1.0 / true_vocab),
        grid_spec=grid_spec,
        out_shape=jax.ShapeDtypeStruct((c1 + _TR, vocab), jnp.float32),
        compiler_params=pltpu.CompilerParams(
            dimension_semantics=("arbitrary",)),
    )


def kernel(batch, alphas, p0, ctx1_keys, ctx1_probs, ctx2_keys, ctx2_probs):
    b, s = batch.shape
    vocab = p0.shape[0]
    c1 = ctx1_keys.shape[0]
    c2 = ctx2_keys.shape[0]
    n = b * s
    flat = batch.reshape(n).astype(jnp.int32)
    al16 = jnp.concatenate([alphas, jnp.zeros((13,), jnp.float32)])

    vp128 = (vocab + 127) // 128 * 128
    padc = vp128 - vocab
    p0p = jnp.pad(p0.reshape(1, vocab), ((0, 0), (0, padc)),
                  constant_values=1.0)
    ctx1p = jnp.pad(ctx1_probs, ((0, 0), (0, padc)), constant_values=1.0)
    table = _make_tc_table(vp128, c1, vocab)(p0p, alphas, ctx1p)

    sc_stage = _make_sc_stage(n, s, vocab, c1, c2)
    out = sc_stage(flat, ctx1_keys, ctx2_keys, table, p0,
                   ctx1_probs, ctx2_probs, al16)
    return out[:, :vocab].reshape(b, s, vocab)


# speedup vs baseline: 2.3587x; 1.3595x over previous
import functools
import math

import jax
import jax.numpy as jnp
from jax import lax
from jax.experimental import pallas as pl
from jax.experimental.pallas import tpu as pltpu
from jax.experimental.pallas import tpu_sc as plsc

_NC = 2
_NS = 16
_NW = _NC * _NS
_L = 16

_INT_MAX = jnp.iinfo(jnp.int32).max
_LN2 = 0.6931471805599453


def _bisect(keys_ref, q, num_keys, steps):
    lo = jnp.zeros((_L,), jnp.int32)
    hi = jnp.full((_L,), num_keys, jnp.int32)
    for _ in range(steps):
        mid = lax.shift_right_arithmetic(lo + hi, 1)
        kv = plsc.load_gather(keys_ref, [mid])
        pred = kv < q
        lo = jnp.where(pred, mid + 1, lo)
        hi = jnp.where(pred, hi, mid)
    idxc = jnp.minimum(lo, num_keys - 1)
    kv = plsc.load_gather(keys_ref, [idxc])
    return idxc, kv == q


def _vlog(x):
    bits = plsc.bitcast(x, jnp.int32)
    e = (lax.shift_right_logical(bits, 23) & 0xFF) - 127
    m = plsc.bitcast((bits & 0x7FFFFF) | 0x3F800000, jnp.float32)
    big = m > (4.0 / 3.0)
    m = jnp.where(big, m * 0.5, m)
    e = jnp.where(big, e + 1, e)
    r = m - 1.0
    s = jnp.float32(-1.0 / 14.0)
    for kk in range(13, 0, -1):
        c = jnp.float32((1.0 if kk % 2 else -1.0) / kk)
        s = s * r + c
    s = s * r
    return e.astype(jnp.float32) * jnp.float32(_LN2) + s


_GC = 16


def _make_sc_stage(n, seq_len, vocab, c1, c2):
    chunk = n // _NW
    steps1 = max(1, math.ceil(math.log2(c1 + 1)))
    steps2 = max(1, math.ceil(math.log2(c2 + 1)))
    nt = (vocab + 127) // 128
    vp128 = nt * 128
    ntm = vocab // 128
    nvec = vp128 // _L
    nch = chunk // _GC
    mesh = plsc.VectorSubcoreMesh(
        core_axis_name="c", subcore_axis_name="s",
        num_cores=_NC, num_subcores=_NS)

    @functools.partial(
        pl.kernel,
        out_type=jax.ShapeDtypeStruct((n * nt, 128), jnp.float32),
        mesh=mesh,
        compiler_params=pltpu.CompilerParams(needs_layout_passes=False),
        scratch_types=[
            pltpu.VMEM((chunk + 8,), jnp.int32),
            pltpu.VMEM((c1 + _L,), jnp.int32),
            pltpu.VMEM((c2 + _L,), jnp.int32),
            pltpu.VMEM((chunk,), jnp.int32),
            pltpu.VMEM((chunk,), jnp.int32),
            pltpu.VMEM((chunk * nt,), jnp.int32),
            pltpu.VMEM((_GC * nt, 128), jnp.float32),
            pltpu.VMEM((_GC * nt, 128), jnp.float32),
            pltpu.VMEM((vp128,), jnp.float32),
            pltpu.VMEM((nt * 8, 128), jnp.float32),
            pltpu.VMEM((nt * 8, 128), jnp.float32),
            pltpu.VMEM((nt, 128), jnp.float32),
            pltpu.VMEM((_L,), jnp.float32),
            pltpu.SemaphoreType.DMA,
            pltpu.SemaphoreType.DMA,
            pltpu.SemaphoreType.DMA,
            pltpu.SemaphoreType.DMA,
            pltpu.SemaphoreType.DMA,
        ],
    )
    def sc_stage(batch_hbm, k1_hbm, k2_hbm, t8_hbm, p0_hbm, p1_hbm, p2_hbm,
                 p1t_hbm, p2t_hbm, al_hbm, out_hbm,
                 qbuf, k1v, k2v, selv, encv, idxall, gbufa, gbufb, p0v,
                 tb1, tb2, rowo, alv, gs0, gs1, ws0, ws1, fsem):
        wid = lax.axis_index("s") * _NC + lax.axis_index("c")
        base = wid * chunk
        pltpu.sync_copy(k1_hbm, k1v.at[pl.ds(0, c1)])
        pltpu.sync_copy(k2_hbm, k2v.at[pl.ds(0, c2)])
        k1v[pl.ds(c1, _L)] = jnp.full((_L,), _INT_MAX, jnp.int32)
        k2v[pl.ds(c2, _L)] = jnp.full((_L,), _INT_MAX, jnp.int32)
        pltpu.sync_copy(al_hbm, alv)
        pltpu.sync_copy(p0_hbm, p0v.at[pl.ds(0, vocab)])
        pltpu.sync_copy(batch_hbm.at[pl.ds(base, chunk)],
                        qbuf.at[pl.ds(8, chunk)])
        @pl.when(wid > 0)
        def _():
            pltpu.sync_copy(batch_hbm.at[pl.ds(base - 8, 8)],
                            qbuf.at[pl.ds(0, 8)])

        iota = lax.iota(jnp.int32, _L)
        c1vec = jnp.full((_L,), c1, jnp.int32)

        def lookup_step(t, carry):
            off = t * _L
            j = lax.rem(base + off, seq_len) + iota
            idxs = off + 8 + iota
            q1 = plsc.load_gather(qbuf, [idxs - 1])
            t2 = plsc.load_gather(qbuf, [idxs - 2])
            i1, hit1 = _bisect(k1v, q1, c1, steps1)
            q2 = t2 * vocab + q1
            i2, hit2 = _bisect(k2v, q2, c2, steps2)
            f1 = hit1 & (j >= 1)
            f2 = hit2 & (j >= 2)
            selv[pl.ds(off, _L)] = jnp.where(f1 & ~f2, i1, c1vec)
            i1p = jnp.where(f1, i1 + 1, 0)
            enc = (lax.shift_left(i2, 12) | lax.shift_left(i1p, 1)
                   | jnp.where(f2, 1, 0))
            encv[pl.ds(off, _L)] = jnp.where(f2, enc, 0)
            lane_sub = iota & (nt - 1)
            lane_pos = lax.shift_right_logical(iota, 3)
            for g in range(nt):
                sv = plsc.load_gather(selv, [off + g * 2 + lane_pos])
                idxall[pl.ds(off * nt + g * _L, _L)] = sv * nt + lane_sub
            return carry

        lax.fori_loop(0, chunk // _L, lookup_step, 0)

        bufs = (gbufa, gbufb)
        gsems = (gs0, gs1)
        wsems = (ws0, ws1)
        rows_per = _GC * nt

        def start_gather(c, b):
            return pltpu.async_copy(
                t8_hbm.at[idxall.at[pl.ds(c * rows_per, rows_per)]],
                bufs[b], gsems[b])

        gh = [None] * nch
        wh = [None] * nch
        gh[0] = start_gather(0, 0)
        for c in range(nch):
            gh[c].wait()
            if c + 1 < nch:
                if c - 1 >= 0:
                    wh[c - 1].wait()
                gh[c + 1] = start_gather(c + 1, (c + 1) % 2)
            wh[c] = pltpu.async_copy(
                bufs[c % 2],
                out_hbm.at[pl.ds(pl.multiple_of((base + c * _GC) * nt, 8),
                                 rows_per)],
                wsems[c % 2])
        if nch >= 2:
            wh[nch - 2].wait()
        wh[nch - 1].wait()

        av = alv[...]
        fzero = jnp.float32(0.0)
        a0 = jnp.max(jnp.where(iota == 0, av, fzero))
        a1 = jnp.max(jnp.where(iota == 1, av, fzero))
        a2 = jnp.max(jnp.where(iota == 2, av, fzero))
        uni = jnp.float32(1.0 / vocab)
        izero = jnp.zeros((_L,), jnp.int32)

        def fetch_band(src, tail_src, band, dst):
            band = pl.multiple_of(band, 8)
            hs = []
            for ct in range(ntm):
                hs.append(pltpu.async_copy(
                    src.at[pl.ds(band, 8), pl.ds(ct * 128, 128)],
                    dst.at[pl.ds(ct * 8, 8)], fsem))
            if ntm < nt:
                hs.append(pltpu.async_copy(
                    tail_src.at[pl.ds(band, 8)],
                    dst.at[pl.ds(ntm * 8, 8)], fsem))
            return hs

        def fix_group(g, carry):
            encg = encv[pl.ds(g * _L, _L)]

            @pl.when(jnp.max(encg) != 0)
            def _():
                def fix_lane(lane, carry2):
                    enc = jnp.max(jnp.where(iota == lane, encg, izero))

                    @pl.when(enc != 0)
                    def _():
                        i1p = lax.shift_right_logical(enc, 1) & 0x7FF
                        i2 = lax.shift_right_logical(enc, 12)
                        i1idx = jnp.maximum(i1p - 1, 0)
                        sub1 = lax.rem(i1idx, 8)
                        sub2 = lax.rem(i2, 8)
                        hs = fetch_band(p1_hbm, p1t_hbm, i1idx - sub1, tb1)
                        hs += fetch_band(p2_hbm, p2t_hbm, i2 - sub2, tb2)
                        for h in hs:
                            h.wait()
                        c1c = jnp.where(i1p > 0, a1, fzero)
                        missc = (a1 - c1c) * uni

                        def vec_step(v, carry3):
                            ct = lax.shift_right_logical(v, 3)
                            part = pl.ds((v & 7) * _L, _L)
                            r1 = tb1[ct * 8 + sub1, part]
                            r2 = tb2[ct * 8 + sub2, part]
                            x = (a0 * p0v[pl.ds(v * _L, _L)] + missc
                                 + c1c * r1 + a2 * r2)
                            rowo[ct, part] = _vlog(x)
                            return carry3

                        lax.fori_loop(0, nvec, vec_step, 0)
                        orow = pl.multiple_of(
                            (base + g * _L + lane) * nt, 8)
                        pltpu.sync_copy(rowo,
                                        out_hbm.at[pl.ds(orow, nt)])

                    return carry2

                lax.fori_loop(0, _L, fix_lane, 0)

            return carry

        lax.fori_loop(0, chunk // _L, fix_group, 0)

    return sc_stage


_TR = 8


def _table_body(p0_ref, al_ref, rows_ref, out_ref, *, nsteps, uni):
    i = pl.program_id(0)
    a0 = al_ref[0]
    a1 = al_ref[1]
    a2 = al_ref[2]
    vp = p0_ref.shape[1]
    base = a0 * p0_ref[0, :] + a2 * uni

    @pl.when(i < nsteps - 1)
    def _():
        vals = jnp.log(base[None, :] + a1 * rows_ref[...])
        out_ref[...] = vals.reshape(_TR, vp // 128, 128).reshape(-1, 128)

    @pl.when(i == nsteps - 1)
    def _():
        row = jnp.log(base + a1 * uni)
        vals = jnp.broadcast_to(row[None, :], (_TR, vp))
        out_ref[...] = vals.reshape(_TR, vp // 128, 128).reshape(-1, 128)


def _make_tc_table(vp, c1, true_vocab):
    nt = vp // 128
    nblk = c1 // _TR
    nsteps = nblk + 1
    grid_spec = pl.GridSpec(
        grid=(nsteps,),
        in_specs=[
            pl.BlockSpec((1, vp), lambda i: (0, 0)),
            pl.BlockSpec(memory_space=pltpu.SMEM),
            pl.BlockSpec((_TR, vp), lambda i: (jnp.minimum(i, nblk - 1), 0)),
        ],
        out_specs=pl.BlockSpec((_TR * nt, 128), lambda i: (i, 0)),
    )
    return pl.pallas_call(
        functools.partial(_table_body, nsteps=nsteps, uni=1.0 / true_vocab),
        grid_spec=grid_spec,
        out_shape=jax.ShapeDtypeStruct(((c1 + _TR) * nt, 128), jnp.float32),
        compiler_params=pltpu.CompilerParams(
            dimension_semantics=("arbitrary",)),
    )


def kernel(batch, alphas, p0, ctx1_keys, ctx1_probs, ctx2_keys, ctx2_probs):
    b, s = batch.shape
    vocab = p0.shape[0]
    c1 = ctx1_keys.shape[0]
    c2 = ctx2_keys.shape[0]
    n = b * s
    flat = batch.reshape(n).astype(jnp.int32)
    al16 = jnp.concatenate([alphas, jnp.zeros((13,), jnp.float32)])

    nt = (vocab + 127) // 128
    vp128 = nt * 128
    padc = vp128 - vocab
    p0p = jnp.pad(p0.reshape(1, vocab), ((0, 0), (0, padc)),
                  constant_values=1.0)
    ctx1p = jnp.pad(ctx1_probs, ((0, 0), (0, padc)), constant_values=1.0)
    table8 = _make_tc_table(vp128, c1, vocab)(p0p, alphas, ctx1p)

    tstart = (vocab // 128) * 128
    tw = vocab - tstart
    if tw > 0:
        p1t = jnp.pad(ctx1_probs[:, tstart:], ((0, 0), (0, 128 - tw)))
        p2t = jnp.pad(ctx2_probs[:, tstart:], ((0, 0), (0, 128 - tw)))
    else:
        p1t = jnp.zeros((c1, 128), jnp.float32)
        p2t = jnp.zeros((c2, 128), jnp.float32)

    sc_stage = _make_sc_stage(n, s, vocab, c1, c2)
    out8 = sc_stage(flat, ctx1_keys, ctx2_keys, table8, p0,
                    ctx1_probs, ctx2_probs, p1t, p2t, al16)
    return out8.reshape(n, vp128)[:, :vocab].reshape(b, s, vocab)
